# Initial kernel scaffold; baseline (speedup 1.0000x reference)
import jax
import jax.numpy as jnp
from jax.experimental import pallas as pl


def kernel(x, table):
    raise NotImplementedError("write your pallas kernel here")


# speedup vs baseline: 2.3986x; 2.3986x over previous
import functools

import jax
import jax.numpy as jnp
from jax import lax
from jax.experimental import pallas as pl
from jax.experimental.pallas import tpu as pltpu
from jax.experimental.pallas import tpu_sc as plsc

_NC = 2
_NS = 16
_NW = _NC * _NS
_LANES = 16

_RND = 12582912.0


def _make_sc_lookup(B, D, per, ch):
    nch = per // ch
    mesh = plsc.VectorSubcoreMesh(core_axis_name="c", subcore_axis_name="s")

    @functools.partial(
        pl.kernel,
        mesh=mesh,
        out_type=jax.ShapeDtypeStruct((B, D), jnp.float32),
        scratch_types=[
            pltpu.VMEM((per,), jnp.float32),
            pltpu.VMEM((ch,), jnp.int32),
            pltpu.VMEM((ch, D), jnp.float32),
            pltpu.SemaphoreType.DMA,
        ],
        compiler_params=pltpu.CompilerParams(use_tc_tiling_on_sc=False),
    )
    def run(x_hbm, tab_hbm, out_hbm, x_v, idx_v, rows_v, sem):
        wid = lax.axis_index("s") * _NC + lax.axis_index("c")
        base = wid * per
        pltpu.sync_copy(x_hbm.at[pl.ds(base, per)], x_v)

        for c in range(nch):
            def tok_body(g, carry, c=c):
                xv = x_v[pl.ds(c * ch + g * _LANES, _LANES)]
                v = jnp.minimum(jnp.maximum(xv, -65.0), 65.0)
                r = (v + _RND) - _RND
                t = r.astype(jnp.int32)
                t = jnp.minimum(jnp.maximum(t, -64), 64) + 65
                t = jnp.where(xv == jnp.inf, 130, t)
                t = jnp.where(xv == -jnp.inf, 0, t)
                t = jnp.where(xv != xv, 0, t)
                idx_v[pl.ds(g * _LANES, _LANES)] = t
                return carry

            lax.fori_loop(0, ch // _LANES, tok_body, 0)

            copies = [
                pltpu.async_copy(
                    tab_hbm.at[idx_v.at[pl.ds(j * 128, 128)]],
                    rows_v.at[pl.ds(j * 128, 128)],
                    sem,
                )
                for j in range(ch // 128)
            ]
            for cp in copies:
                cp.wait()
            pltpu.sync_copy(rows_v, out_hbm.at[pl.ds(base + c * ch, ch)])

    return run


def kernel(x, table):
    lead = x.shape[:-1]
    xf = x.reshape(-1)
    B = xf.shape[0]
    D = table.shape[1]
    per = B // _NW
    ch = 1024
    out = _make_sc_lookup(B, D, per, ch)(xf, table)
    return out.reshape(*lead[:-1], lead[-1], D)
